# Initial kernel scaffold; baseline (speedup 1.0000x reference)
import jax
import jax.numpy as jnp
from jax.experimental import pallas as pl


def kernel(idx, core0, core1, core2, core3):
    raise NotImplementedError("write your pallas kernel here")


# speedup vs baseline: 6.2389x; 6.2389x over previous
import jax
import jax.numpy as jnp
from jax import lax
from jax.experimental import pallas as pl
from jax.experimental.pallas import tpu as pltpu
from jax.experimental.pallas import tpu_sc as plsc

R = 8
L = 16


def _build_sc_call(B, n):
    NW = 32
    BP = B // (NW // 2)
    n_grp = BP // L
    mesh = plsc.VectorSubcoreMesh(core_axis_name="c", subcore_axis_name="s")

    def body(t0, t1, t2, t3, i0, i1, i2, i3, out,
             tab_small, tab_big, idx_a, idx_b, vbuf, out_v, stage, sem):
        c = lax.axis_index("c")
        s = lax.axis_index("s")
        k = s // 2
        base = (c * 8 + k) * BP
        role_a = (s % 2) == 0

        iota = lax.iota(jnp.int32, L)

        @pl.when(role_a)
        def _phase_a():
            pltpu.sync_copy(t0, tab_small)
            pltpu.sync_copy(t1, tab_big)
            pltpu.sync_copy(i0.at[pl.ds(base, BP)], idx_a)
            pltpu.sync_copy(i1.at[pl.ds(base, BP)], idx_b)

            def group(g, carry):
                o = g * L
                b0 = idx_a[pl.ds(o, L)] * R
                b1 = idx_b[pl.ds(o, L)] * (R * R)
                v = [plsc.load_gather(tab_small, [b0 + i]) for i in range(R)]
                for j in range(R):
                    acc = v[0] * plsc.load_gather(tab_big, [b1 + j])
                    for i in range(1, R):
                        acc = acc + v[i] * plsc.load_gather(
                            tab_big, [b1 + (R * i + j)])
                    vbuf[pl.ds(j * BP + o, L)] = acc
                return carry

            lax.fori_loop(0, n_grp, group, 0)
            pltpu.sync_copy(vbuf, stage.at[k])

        @pl.when(jnp.logical_not(role_a))
        def _phase_b_load():
            pltpu.sync_copy(t3, tab_small)
            pltpu.sync_copy(t2, tab_big)
            pltpu.sync_copy(i2.at[pl.ds(base, BP)], idx_a)
            pltpu.sync_copy(i3.at[pl.ds(base, BP)], idx_b)

        plsc.subcore_barrier()

        @pl.when(jnp.logical_not(role_a))
        def _phase_b():
            pltpu.sync_copy(stage.at[k], vbuf)

            def group(g, carry):
                o = g * L
                b2 = idx_a[pl.ds(o, L)] * (R * R)
                b3 = idx_b[pl.ds(o, L)] * R
                v = [vbuf[pl.ds(i * BP + o, L)] for i in range(R)]
                w = []
                for j in range(R):
                    acc = v[0] * plsc.load_gather(tab_big, [b2 + j])
                    for i in range(1, R):
                        acc = acc + v[i] * plsc.load_gather(
                            tab_big, [b2 + (R * i + j)])
                    w.append(acc)
                res = w[0] * plsc.load_gather(tab_small, [b3])
                for j in range(1, R):
                    res = res + w[j] * plsc.load_gather(tab_small, [b3 + j])
                out_v[pl.ds(o, L)] = res
                return carry

            lax.fori_loop(0, n_grp, group, 0)
            pltpu.sync_copy(out_v, out.at[pl.ds(base, BP)])

    return pl.kernel(
        body,
        mesh=mesh,
        compiler_params=pltpu.CompilerParams(needs_layout_passes=False),
        out_type=jax.ShapeDtypeStruct((B,), jnp.float32),
        scratch_types=[
            pltpu.VMEM((n * R,), jnp.float32),
            pltpu.VMEM((n * R * R,), jnp.float32),
            pltpu.VMEM((BP,), jnp.int32),
            pltpu.VMEM((BP,), jnp.int32),
            pltpu.VMEM((BP * R,), jnp.float32),
            pltpu.VMEM((BP,), jnp.float32),
            pltpu.VMEM_SHARED((8, BP * R), jnp.float32),
            pltpu.SemaphoreType.DMA,
        ],
    )


def kernel(idx, core0, core1, core2, core3):
    n = core1.shape[1]
    B = idx.shape[0]

    t0 = jnp.transpose(core0, (1, 0, 2)).reshape(n * R)
    t1 = jnp.transpose(core1, (1, 0, 2)).reshape(n * R * R)
    t2 = jnp.transpose(core2, (1, 0, 2)).reshape(n * R * R)
    t3 = jnp.transpose(core3, (1, 0, 2)).reshape(n * R)
    idx_t = idx.astype(jnp.int32).T
    i0, i1, i2, i3 = idx_t[0], idx_t[1], idx_t[2], idx_t[3]

    fn = _build_sc_call(B, n)
    return fn(t0, t1, t2, t3, i0, i1, i2, i3)


# speedup vs baseline: 13.3513x; 2.1400x over previous
import jax
import jax.numpy as jnp
from jax import lax
from jax.experimental import pallas as pl
from jax.experimental.pallas import tpu as pltpu
from jax.experimental.pallas import tpu_sc as plsc

R = 8
L = 16
SB = R * R + 1
SS = R + 1


def _build_sc_call(B, n):
    NW = 32
    BP = B // (NW // 2)
    n_grp = BP // L
    mesh = plsc.VectorSubcoreMesh(core_axis_name="c", subcore_axis_name="s")

    def body(t0, t1, t2, t3, i0, i1, i2, i3, out,
             tab_small, tab_big, idx_a, idx_b, vbuf, out_v, stage, sem):
        c = lax.axis_index("c")
        s = lax.axis_index("s")
        k = s // 2
        base = (c * 8 + k) * BP
        role_a = (s % 2) == 0

        iota = lax.iota(jnp.int32, L)

        @pl.when(role_a)
        def _phase_a():
            pltpu.sync_copy(t0, tab_small)
            pltpu.sync_copy(t1, tab_big)
            pltpu.sync_copy(i0.at[pl.ds(base, BP)], idx_a)
            pltpu.sync_copy(i1.at[pl.ds(base, BP)], idx_b)

            def group(g, carry):
                o = g * L
                b0 = idx_a[pl.ds(o, L)] * SS
                b1 = idx_b[pl.ds(o, L)] * SB
                v = [plsc.load_gather(tab_small, [b0 + i]) for i in range(R)]
                for j in range(R):
                    acc = v[0] * plsc.load_gather(tab_big, [b1 + j])
                    for i in range(1, R):
                        acc = acc + v[i] * plsc.load_gather(
                            tab_big, [b1 + (R * i + j)])
                    vbuf[pl.ds(j * BP + o, L)] = acc
                return carry

            lax.fori_loop(0, n_grp, group, 0)
            pltpu.sync_copy(vbuf, stage.at[k])

        @pl.when(jnp.logical_not(role_a))
        def _phase_b_load():
            pltpu.sync_copy(t3, tab_small)
            pltpu.sync_copy(t2, tab_big)
            pltpu.sync_copy(i2.at[pl.ds(base, BP)], idx_a)
            pltpu.sync_copy(i3.at[pl.ds(base, BP)], idx_b)

        plsc.subcore_barrier()

        @pl.when(jnp.logical_not(role_a))
        def _phase_b():
            pltpu.sync_copy(stage.at[k], vbuf)

            def group(g, carry):
                o = g * L
                b2 = idx_a[pl.ds(o, L)] * SB
                b3 = idx_b[pl.ds(o, L)] * SS
                v = [vbuf[pl.ds(i * BP + o, L)] for i in range(R)]
                w = []
                for j in range(R):
                    acc = v[0] * plsc.load_gather(tab_big, [b2 + j])
                    for i in range(1, R):
                        acc = acc + v[i] * plsc.load_gather(
                            tab_big, [b2 + (R * i + j)])
                    w.append(acc)
                res = w[0] * plsc.load_gather(tab_small, [b3])
                for j in range(1, R):
                    res = res + w[j] * plsc.load_gather(tab_small, [b3 + j])
                out_v[pl.ds(o, L)] = res
                return carry

            lax.fori_loop(0, n_grp, group, 0)
            pltpu.sync_copy(out_v, out.at[pl.ds(base, BP)])

    return pl.kernel(
        body,
        mesh=mesh,
        compiler_params=pltpu.CompilerParams(needs_layout_passes=False),
        out_type=jax.ShapeDtypeStruct((B,), jnp.float32),
        scratch_types=[
            pltpu.VMEM((n * SS,), jnp.float32),
            pltpu.VMEM((n * SB,), jnp.float32),
            pltpu.VMEM((BP,), jnp.int32),
            pltpu.VMEM((BP,), jnp.int32),
            pltpu.VMEM((BP * R,), jnp.float32),
            pltpu.VMEM((BP,), jnp.float32),
            pltpu.VMEM_SHARED((8, BP * R), jnp.float32),
            pltpu.SemaphoreType.DMA,
        ],
    )


def kernel(idx, core0, core1, core2, core3):
    n = core1.shape[1]
    B = idx.shape[0]

    pad_s = ((0, 0), (0, SS - R))
    pad_b = ((0, 0), (0, SB - R * R))
    t0 = jnp.pad(jnp.transpose(core0, (1, 0, 2)).reshape(n, R), pad_s).reshape(n * SS)
    t1 = jnp.pad(jnp.transpose(core1, (1, 0, 2)).reshape(n, R * R), pad_b).reshape(n * SB)
    t2 = jnp.pad(jnp.transpose(core2, (1, 0, 2)).reshape(n, R * R), pad_b).reshape(n * SB)
    t3 = jnp.pad(jnp.transpose(core3, (1, 0, 2)).reshape(n, R), pad_s).reshape(n * SS)
    idx_t = idx.astype(jnp.int32).T
    i0, i1, i2, i3 = idx_t[0], idx_t[1], idx_t[2], idx_t[3]

    fn = _build_sc_call(B, n)
    return fn(t0, t1, t2, t3, i0, i1, i2, i3)
